# baseline (speedup 1.0000x reference)
import functools

import jax
import jax.numpy as jnp
from jax import lax
from jax.experimental import pallas as pl
from jax.experimental.pallas import tpu as pltpu
from jax.experimental.pallas import tpu_sc as plsc

N = 10000
E = 320000
D_IN = 128
HID = 128
EMB = 16

NC = 2
NS = 16
NW = NC * NS
EPW = E // NW
CHUNK = 80
NCH = EPW // CHUNK
NP = 10240
RPT = NP // NS
DCH = 2048


def _pre_body(x_ref, wg_ref, al_ref, ar_ref, h_ref, a1_ref, a2_ref):
    h = jnp.dot(x_ref[...], wg_ref[...], preferred_element_type=jnp.float32)
    h_ref[...] = h
    a1_ref[...] = jnp.sum(h * al_ref[...], axis=1, keepdims=True)
    a2_ref[...] = jnp.sum(h * ar_ref[...], axis=1, keepdims=True)


def _pre_call(x, wg, al, ar):
    return pl.pallas_call(
        _pre_body,
        out_shape=[
            jax.ShapeDtypeStruct((N, HID), jnp.float32),
            jax.ShapeDtypeStruct((N, 1), jnp.float32),
            jax.ShapeDtypeStruct((N, 1), jnp.float32),
        ],
    )(x, wg, al, ar)


_sc_mesh = plsc.VectorSubcoreMesh(core_axis_name="c", subcore_axis_name="s")


@functools.partial(
    pl.kernel,
    out_type=(
        jax.ShapeDtypeStruct((NC, NP, HID), jnp.float32),
        jax.ShapeDtypeStruct((NC, NP), jnp.float32),
    ),
    mesh=_sc_mesh,
    scratch_types=[
        pltpu.VMEM((NP,), jnp.float32),
        pltpu.VMEM((N,), jnp.float32),
        pltpu.VMEM((4 * CHUNK,), jnp.int32),
        pltpu.VMEM((4 * CHUNK,), jnp.int32),
        pltpu.VMEM((1, CHUNK), jnp.int32),
        pltpu.VMEM((1, CHUNK), jnp.int32),
        pltpu.VMEM((1, CHUNK), jnp.float32),
        pltpu.VMEM((1, CHUNK), jnp.float32),
        pltpu.VMEM((CHUNK, HID), jnp.float32),
        pltpu.VMEM((CHUNK, HID), jnp.float32),
        pltpu.VMEM_SHARED((NP, HID), jnp.float32),
        pltpu.VMEM_SHARED((NP,), jnp.float32),
        pltpu.SemaphoreType.DMA,
        pltpu.SemaphoreType.DMA,
        pltpu.SemaphoreType.DMA,
        pltpu.SemaphoreType.DMA,
        pltpu.SemaphoreType.DMA,
        pltpu.SemaphoreType.DMA,
        pltpu.SemaphoreType.DMA,
        pltpu.SemaphoreType.DMA,
    ],
    compiler_params=pltpu.CompilerParams(needs_layout_passes=False),
)
def _sc_edge(sd_hbm, a1_hbm, a2_hbm, h_hbm, num_out, den_out,
             a1_v, a2_v, sdc0, sdc1, dstx0, dstx1, wc0, wc1, rows0, rows1,
             num_sh, den_sh, gsem0, gsem1, isem0, isem1,
             nsem0, nsem1, dsem0, dsem1):
    c = lax.axis_index("c")
    s = lax.axis_index("s")
    wid = c * NS + s
    zf = jnp.zeros((16,), jnp.float32)

    bufs = ((sdc0, rows0, dstx0, wc0, gsem0, isem0, nsem0, dsem0),
            (sdc1, rows1, dstx1, wc1, gsem1, isem1, nsem1, dsem1))

    ebase = wid * EPW
    PCH = 2 * CHUNK

    def _pidx_start(p, q):
        sdc = bufs[q][0]
        pltpu.async_copy(sd_hbm.at[pl.ds(ebase + p * PCH, PCH)],
                         sdc.at[pl.ds(0, PCH)], bufs[q][5])
        pltpu.async_copy(sd_hbm.at[pl.ds(E + ebase + p * PCH, PCH)],
                         sdc.at[pl.ds(PCH, PCH)], bufs[q][5])

    def _pidx_start_last(q):
        sdc = bufs[q][0]
        pltpu.async_copy(sd_hbm.at[pl.ds(ebase + EPW - CHUNK, CHUNK)],
                         sdc.at[pl.ds(0, CHUNK)], bufs[q][5])
        pltpu.async_copy(sd_hbm.at[pl.ds(E + ebase + EPW - CHUNK, CHUNK)],
                         sdc.at[pl.ds(PCH, CHUNK)], bufs[q][5])

    def _pidx_wait(q, full=True):
        sdc = bufs[q][0]
        n = PCH if full else CHUNK
        pltpu.make_async_copy(sd_hbm.at[pl.ds(0, n)],
                              sdc.at[pl.ds(0, n)], bufs[q][5]).wait()
        pltpu.make_async_copy(sd_hbm.at[pl.ds(0, n)],
                              sdc.at[pl.ds(PCH, n)], bufs[q][5]).wait()

    def _gather_start(q, b):
        sdc, rows = bufs[q][0], bufs[b][1]
        pltpu.async_copy(h_hbm.at[sdc.at[pl.ds(b * CHUNK, CHUNK)]],
                         rows, bufs[b][4])

    def _gather_wait(q, b):
        sdc, rows = bufs[q][0], bufs[b][1]
        pltpu.make_async_copy(h_hbm.at[sdc.at[pl.ds(b * CHUNK, CHUNK)]],
                              rows, bufs[b][4]).wait()

    def _nscat_wait(b):
        _, rows, dstx, _, _, _, nsem, _ = bufs[b]
        pltpu.make_async_copy(rows, num_sh.at[dstx.at[0]], nsem).wait()

    def _dscat_wait(b):
        _, _, dstx, wc, _, _, _, dsem = bufs[b]
        pltpu.make_async_copy(wc.at[0], den_sh.at[dstx.at[0]], dsem).wait()

    def _process(ii, q, b, steady):
        sdc = bufs[q][0]
        _, rows, dstx, wc, _, _, nsem, dsem = bufs[b]
        nb = 1 - b
        if steady:
            @pl.when(ii > 0)
            def _():
                _nscat_wait(nb)

            _gather_start(q if b == 0 else 1 - q, nb)

        @pl.when(ii > 1)
        def _():
            _dscat_wait(b)

        for j in range(CHUNK // 16):
            s16 = sdc[pl.ds(b * CHUNK + j * 16, 16)]
            d16 = sdc[pl.ds(PCH + b * CHUNK + j * 16, 16)]
            e = plsc.load_gather(a1_v, [s16]) + plsc.load_gather(a2_v, [d16])
            e = jnp.where(e >= 0, e, 0.2 * e)
            wc[0, pl.ds(j * 16, 16)] = jnp.exp(e)

        for j in range(CHUNK // 16):
            dstx[0, pl.ds(j * 16, 16)] = sdc[pl.ds(PCH + b * CHUNK + j * 16, 16)]

        _gather_wait(q, b)

        @plsc.parallel_loop(0, CHUNK // 16, unroll=2)
        def _scale(g):
            w16 = wc[0, pl.ds(g * 16, 16)]
            base = g * 16
            for t in range(16):
                wk = jnp.full((16,), w16[t])
                for j in range(HID // 16):
                    rows[base + t, pl.ds(j * 16, 16)] = (
                        rows[base + t, pl.ds(j * 16, 16)] * wk)

        pltpu.async_copy(rows, num_sh.at[dstx.at[0]], nsem, add=True)
        pltpu.async_copy(wc.at[0], den_sh.at[dstx.at[0]], dsem, add=True)

    _pidx_start(0, 0)

    def _zrow(k, _):
        for j in range(HID // 16):
            rows1[k, pl.ds(j * 16, 16)] = zf
        return 0

    lax.fori_loop(0, CHUNK, _zrow, 0)

    def _za(k, _):
        a1_v[pl.ds(k * 16, 16)] = zf
        return 0

    lax.fori_loop(0, NP // 16, _za, 0)

    for t in range(RPT // CHUNK):
        pltpu.async_copy(rows1, num_sh.at[pl.ds(s * RPT + t * CHUNK, CHUNK)],
                         dsem0)

    @pl.when(s < NP // DCH)
    def _():
        pltpu.async_copy(a1_v.at[pl.ds(0, DCH)],
                         den_sh.at[pl.ds(s * DCH, DCH)], dsem0)

    for t in range(RPT // CHUNK):
        pltpu.make_async_copy(
            rows1, num_sh.at[pl.ds(s * RPT + t * CHUNK, CHUNK)], dsem0).wait()

    @pl.when(s < NP // DCH)
    def _():
        pltpu.make_async_copy(a1_v.at[pl.ds(0, DCH)],
                              den_sh.at[pl.ds(s * DCH, DCH)], dsem0).wait()

    pltpu.async_copy(a1_hbm, a1_v.at[pl.ds(0, N)], dsem0)
    pltpu.async_copy(a2_hbm, a2_v, dsem1)
    pltpu.make_async_copy(a1_hbm, a1_v.at[pl.ds(0, N)], dsem0).wait()
    pltpu.make_async_copy(a2_hbm, a2_v, dsem1).wait()

    _pidx_wait(0)
    _pidx_start(1, 1)
    _gather_start(0, 0)

    plsc.subcore_barrier()

    NPAIR = (NCH - 1) // 2
    NQ = NPAIR // 2

    def _quad(t, _):
        pa = 2 * t
        _process(2 * pa, 0, 0, True)
        _pidx_wait(1)
        _process(2 * pa + 1, 0, 1, True)

        @pl.when(t < NQ - 1)
        def _():
            _pidx_start(pa + 2, 0)

        @pl.when(t == NQ - 1)
        def _():
            _pidx_start_last(0)

        pb = 2 * t + 1
        _process(2 * pb, 1, 0, True)

        @pl.when(t < NQ - 1)
        def _():
            _pidx_wait(0)

        @pl.when(t == NQ - 1)
        def _():
            _pidx_wait(0, full=False)

        _process(2 * pb + 1, 1, 1, True)

        @pl.when(t < NQ - 1)
        def _():
            _pidx_start(pb + 2, 1)
        return 0

    lax.fori_loop(0, NQ, _quad, 0)
    _process(NCH - 1, 0, 0, False)

    _nscat_wait(1)
    _nscat_wait(0)
    _dscat_wait(1)
    _dscat_wait(0)

    plsc.subcore_barrier()

    pltpu.sync_copy(num_sh.at[pl.ds(s * RPT, RPT)],
                    num_out.at[c, pl.ds(s * RPT, RPT)])

    @pl.when(s < NP // DCH)
    def _():
        pltpu.sync_copy(den_sh.at[pl.ds(s * DCH, DCH)],
                        den_out.at[c, pl.ds(s * DCH, DCH)])


BLK = 2000
EW = 3 * EMB + 1


def _post_body(num_ref, den_ref, hp1_ref, hp2_ref, bg_ref,
               wzh1_ref, bzh1_ref, wzh2_ref, bzh2_ref,
               wef_ref, bef_ref, c_ref, tp_ref):
    num = num_ref[0] + num_ref[1]
    den = den_ref[0, :, 0] + den_ref[1, :, 0]
    out = num / (den[:, None] + 1e-16) + bg_ref[...]
    zh1 = (jnp.dot(out, wzh1_ref[...], preferred_element_type=jnp.float32)
           + bzh1_ref[...])
    z1 = jax.nn.sigmoid(zh1[:, :HID])
    ht1 = jnp.tanh(zh1[:, HID:])
    h1 = (1.0 - z1) * hp1_ref[...] + z1 * ht1
    zh2 = (jnp.dot(h1, wzh2_ref[...], preferred_element_type=jnp.float32)
           + bzh2_ref[...])
    z2 = jax.nn.sigmoid(zh2[:, :HID])
    ht2 = jnp.tanh(zh2[:, HID:])
    h2 = (1.0 - z2) * hp2_ref[...] + z2 * ht2
    ef = (jnp.dot(h2, wef_ref[...], preferred_element_type=jnp.float32)
          + bef_ref[...])
    c_ref[...] = ef[:, :3 * EMB]
    tp_ref[...] = ef[:, 3 * EMB:]


def _post_call(num, den3, hp1, hp2, bg, wzh1, bzh1, wzh2, bzh2, wef, bef):
    full = lambda shape: pl.BlockSpec(shape, lambda i: (0,) * len(shape))
    return pl.pallas_call(
        _post_body,
        grid=(N // BLK,),
        in_specs=[
            pl.BlockSpec((NC, BLK, HID), lambda i: (0, i, 0)),
            pl.BlockSpec((NC, BLK, 1), lambda i: (0, i, 0)),
            pl.BlockSpec((BLK, HID), lambda i: (i, 0)),
            pl.BlockSpec((BLK, HID), lambda i: (i, 0)),
            full((1, HID)),
            full((HID, 2 * HID)), full((1, 2 * HID)),
            full((HID, 2 * HID)), full((1, 2 * HID)),
            full((HID, EW)), full((1, EW)),
        ],
        out_specs=[
            pl.BlockSpec((BLK, 3 * EMB), lambda i: (i, 0)),
            pl.BlockSpec((BLK, 1), lambda i: (i, 0)),
        ],
        out_shape=[
            jax.ShapeDtypeStruct((N, 3 * EMB), jnp.float32),
            jax.ShapeDtypeStruct((N, 1), jnp.float32),
        ],
    )(num, den3, hp1, hp2, bg, wzh1, bzh1, wzh2, bzh2, wef, bef)


def kernel(x, edge_index, h_prev1, h_prev2, W_gat, attn_l, attn_r, b_gat,
           Wz1, bz1, Wh1, bh1, Wz2, bz2, Wh2, bh2,
           We1, be1, We2, be2, We3, be3, Wf, bf):
    al = attn_l.reshape(1, HID)
    ar = attn_r.reshape(1, HID)
    h, a1k, a2k = _pre_call(x, W_gat, al, ar)

    num, den = _sc_edge(edge_index.reshape(2 * E), a1k.reshape(N),
                        a2k.reshape(N), h)

    wzh1 = jnp.concatenate([Wz1, Wh1], axis=1)
    bzh1 = jnp.concatenate([bz1, bh1]).reshape(1, 2 * HID)
    wzh2 = jnp.concatenate([Wz2, Wh2], axis=1)
    bzh2 = jnp.concatenate([bz2, bh2]).reshape(1, 2 * HID)
    wef = jnp.concatenate([We1, We2, We3, Wf], axis=1)
    bef = jnp.concatenate([be1, be2, be3, bf]).reshape(1, EW)
    cat, tp = _post_call(
        num, den.reshape(NC, NP, 1), h_prev1, h_prev2, b_gat.reshape(1, HID),
        wzh1, bzh1, wzh2, bzh2, wef, bef)
    return (cat.reshape(N, 3, EMB), tp.reshape(N))


# speedup vs baseline: 1.0976x; 1.0976x over previous
import functools

import jax
import jax.numpy as jnp
from jax import lax
from jax.experimental import pallas as pl
from jax.experimental.pallas import tpu as pltpu
from jax.experimental.pallas import tpu_sc as plsc

N = 10000
E = 320000
D_IN = 128
HID = 128
EMB = 16

NC = 2
NS = 16
NW = NC * NS
EPW = E // NW
CHUNK = 80
NCH = EPW // CHUNK
NP = 10240
RPT = NP // NS
DCH = 2048


def _pre_body(x_ref, wg_ref, al_ref, ar_ref, h_ref, a1_ref, a2_ref):
    h = jnp.dot(x_ref[...], wg_ref[...], preferred_element_type=jnp.float32)
    h_ref[...] = h
    a1_ref[...] = jnp.sum(h * al_ref[...], axis=1, keepdims=True)
    a2_ref[...] = jnp.sum(h * ar_ref[...], axis=1, keepdims=True)


def _pre_call(x, wg, al, ar):
    return pl.pallas_call(
        _pre_body,
        out_shape=[
            jax.ShapeDtypeStruct((N, HID), jnp.float32),
            jax.ShapeDtypeStruct((N, 1), jnp.float32),
            jax.ShapeDtypeStruct((N, 1), jnp.float32),
        ],
    )(x, wg, al, ar)


_sc_mesh = plsc.VectorSubcoreMesh(core_axis_name="c", subcore_axis_name="s")


@functools.partial(
    pl.kernel,
    out_type=(
        jax.ShapeDtypeStruct((NC, NP, HID), jnp.float32),
        jax.ShapeDtypeStruct((NC, NP), jnp.float32),
    ),
    mesh=_sc_mesh,
    scratch_types=[
        pltpu.VMEM((NP,), jnp.float32),
        pltpu.VMEM((N,), jnp.float32),
        pltpu.VMEM((2, CHUNK), jnp.int32),
        pltpu.VMEM((2, CHUNK), jnp.int32),
        pltpu.VMEM((1, CHUNK), jnp.int32),
        pltpu.VMEM((1, CHUNK), jnp.int32),
        pltpu.VMEM((1, CHUNK), jnp.float32),
        pltpu.VMEM((1, CHUNK), jnp.float32),
        pltpu.VMEM((CHUNK, HID), jnp.float32),
        pltpu.VMEM((CHUNK, HID), jnp.float32),
        pltpu.VMEM_SHARED((NP, HID), jnp.float32),
        pltpu.VMEM_SHARED((NP,), jnp.float32),
        pltpu.SemaphoreType.DMA,
        pltpu.SemaphoreType.DMA,
        pltpu.SemaphoreType.DMA,
        pltpu.SemaphoreType.DMA,
        pltpu.SemaphoreType.DMA,
        pltpu.SemaphoreType.DMA,
        pltpu.SemaphoreType.DMA,
        pltpu.SemaphoreType.DMA,
    ],
    compiler_params=pltpu.CompilerParams(needs_layout_passes=False),
)
def _sc_edge(sd_hbm, a1_hbm, a2_hbm, h_hbm, num_out, den_out,
             a1_v, a2_v, sdc0, sdc1, dstx0, dstx1, wc0, wc1, rows0, rows1,
             num_sh, den_sh, gsem0, gsem1, isem0, isem1,
             nsem0, nsem1, dsem0, dsem1):
    c = lax.axis_index("c")
    s = lax.axis_index("s")
    wid = c * NS + s
    zf = jnp.zeros((16,), jnp.float32)

    bufs = ((sdc0, rows0, dstx0, wc0, gsem0, isem0, nsem0, dsem0),
            (sdc1, rows1, dstx1, wc1, gsem1, isem1, nsem1, dsem1))

    ebase = wid * EPW

    def _idx_start(ii, b):
        sdc, _, _, _, _, isem, _, _ = bufs[b]
        pltpu.async_copy(sd_hbm.at[pl.ds(ebase + ii * CHUNK, CHUNK)],
                         sdc.at[0], isem)
        pltpu.async_copy(sd_hbm.at[pl.ds(E + ebase + ii * CHUNK, CHUNK)],
                         sdc.at[1], isem)

    def _idx_wait(b):
        sdc, _, _, _, _, isem, _, _ = bufs[b]
        pltpu.make_async_copy(sd_hbm.at[pl.ds(0, CHUNK)], sdc.at[0], isem).wait()
        pltpu.make_async_copy(sd_hbm.at[pl.ds(0, CHUNK)], sdc.at[1], isem).wait()

    def _gather_start(b):
        sdc, rows, _, _, gsem, _, _, _ = bufs[b]
        pltpu.async_copy(h_hbm.at[sdc.at[0]], rows, gsem)

    def _gather_wait(b):
        sdc, rows, _, _, gsem, _, _, _ = bufs[b]
        pltpu.make_async_copy(h_hbm.at[sdc.at[0]], rows, gsem).wait()

    def _nscat_wait(b):
        _, rows, dstx, _, _, _, nsem, _ = bufs[b]
        pltpu.make_async_copy(rows, num_sh.at[dstx.at[0]], nsem).wait()

    def _dscat_wait(b):
        _, _, dstx, wc, _, _, _, dsem = bufs[b]
        pltpu.make_async_copy(wc.at[0], den_sh.at[dstx.at[0]], dsem).wait()

    def _process(ii, b, steady):
        sdc, rows, dstx, wc, _, _, nsem, dsem = bufs[b]
        nb = 1 - b
        if steady:
            _idx_wait(nb)

            @pl.when(ii > 0)
            def _():
                _nscat_wait(nb)

            _gather_start(nb)

        @pl.when(ii > 1)
        def _():
            _dscat_wait(b)

        for j in range(CHUNK // 16):
            s16 = sdc[0, pl.ds(j * 16, 16)]
            d16 = sdc[1, pl.ds(j * 16, 16)]
            e = plsc.load_gather(a1_v, [s16]) + plsc.load_gather(a2_v, [d16])
            e = jnp.where(e >= 0, e, 0.2 * e)
            wc[0, pl.ds(j * 16, 16)] = jnp.exp(e)

        for j in range(CHUNK // 16):
            dstx[0, pl.ds(j * 16, 16)] = sdc[1, pl.ds(j * 16, 16)]

        _gather_wait(b)

        @plsc.parallel_loop(0, CHUNK // 16, unroll=2)
        def _scale(g):
            w16 = wc[0, pl.ds(g * 16, 16)]
            base = g * 16
            for t in range(16):
                wk = jnp.full((16,), w16[t])
                for j in range(HID // 16):
                    rows[base + t, pl.ds(j * 16, 16)] = (
                        rows[base + t, pl.ds(j * 16, 16)] * wk)

        pltpu.async_copy(rows, num_sh.at[dstx.at[0]], nsem, add=True)
        pltpu.async_copy(wc.at[0], den_sh.at[dstx.at[0]], dsem, add=True)

        if steady:
            @pl.when(ii < NCH - 2)
            def _():
                _idx_start(ii + 2, b)

    _idx_start(0, 0)
    _idx_start(1, 1)

    def _zrow(k, _):
        for j in range(HID // 16):
            rows1[k, pl.ds(j * 16, 16)] = zf
        return 0

    lax.fori_loop(0, CHUNK, _zrow, 0)

    def _za(k, _):
        a1_v[pl.ds(k * 16, 16)] = zf
        return 0

    lax.fori_loop(0, NP // 16, _za, 0)

    for t in range(RPT // CHUNK):
        pltpu.async_copy(rows1, num_sh.at[pl.ds(s * RPT + t * CHUNK, CHUNK)],
                         dsem0)

    @pl.when(s < NP // DCH)
    def _():
        pltpu.async_copy(a1_v.at[pl.ds(0, DCH)],
                         den_sh.at[pl.ds(s * DCH, DCH)], dsem0)

    for t in range(RPT // CHUNK):
        pltpu.make_async_copy(
            rows1, num_sh.at[pl.ds(s * RPT + t * CHUNK, CHUNK)], dsem0).wait()

    @pl.when(s < NP // DCH)
    def _():
        pltpu.make_async_copy(a1_v.at[pl.ds(0, DCH)],
                              den_sh.at[pl.ds(s * DCH, DCH)], dsem0).wait()

    pltpu.async_copy(a1_hbm, a1_v.at[pl.ds(0, N)], dsem0)
    pltpu.async_copy(a2_hbm, a2_v, dsem1)
    pltpu.make_async_copy(a1_hbm, a1_v.at[pl.ds(0, N)], dsem0).wait()
    pltpu.make_async_copy(a2_hbm, a2_v, dsem1).wait()

    _idx_wait(0)
    _gather_start(0)

    plsc.subcore_barrier()

    def _pair(t, _):
        _process(2 * t, 0, True)
        _process(2 * t + 1, 1, True)
        return 0

    lax.fori_loop(0, (NCH - 1) // 2, _pair, 0)
    _process(NCH - 1, 0, False)

    _nscat_wait(1)
    _nscat_wait(0)
    _dscat_wait(1)
    _dscat_wait(0)

    plsc.subcore_barrier()

    pltpu.sync_copy(num_sh.at[pl.ds(s * RPT, RPT)],
                    num_out.at[c, pl.ds(s * RPT, RPT)])

    @pl.when(s < NP // DCH)
    def _():
        pltpu.sync_copy(den_sh.at[pl.ds(s * DCH, DCH)],
                        den_out.at[c, pl.ds(s * DCH, DCH)])


BLK = 2000
EW = 3 * EMB + 1


def _post_body(num_ref, den_ref, hp1_ref, hp2_ref, bg_ref,
               wzh1_ref, bzh1_ref, wzh2_ref, bzh2_ref,
               wef_ref, bef_ref, c_ref, tp_ref):
    num = num_ref[0] + num_ref[1]
    den = den_ref[0, :, 0] + den_ref[1, :, 0]
    out = num / (den[:, None] + 1e-16) + bg_ref[...]
    zh1 = (jnp.dot(out, wzh1_ref[...], preferred_element_type=jnp.float32)
           + bzh1_ref[...])
    z1 = jax.nn.sigmoid(zh1[:, :HID])
    ht1 = jnp.tanh(zh1[:, HID:])
    h1 = (1.0 - z1) * hp1_ref[...] + z1 * ht1
    zh2 = (jnp.dot(h1, wzh2_ref[...], preferred_element_type=jnp.float32)
           + bzh2_ref[...])
    z2 = jax.nn.sigmoid(zh2[:, :HID])
    ht2 = jnp.tanh(zh2[:, HID:])
    h2 = (1.0 - z2) * hp2_ref[...] + z2 * ht2
    ef = (jnp.dot(h2, wef_ref[...], preferred_element_type=jnp.float32)
          + bef_ref[...])
    c_ref[...] = ef[:, :3 * EMB]
    tp_ref[...] = ef[:, 3 * EMB:]


def _post_call(num, den3, hp1, hp2, bg, wzh1, bzh1, wzh2, bzh2, wef, bef):
    full = lambda shape: pl.BlockSpec(shape, lambda i: (0,) * len(shape))
    return pl.pallas_call(
        _post_body,
        grid=(N // BLK,),
        in_specs=[
            pl.BlockSpec((NC, BLK, HID), lambda i: (0, i, 0)),
            pl.BlockSpec((NC, BLK, 1), lambda i: (0, i, 0)),
            pl.BlockSpec((BLK, HID), lambda i: (i, 0)),
            pl.BlockSpec((BLK, HID), lambda i: (i, 0)),
            full((1, HID)),
            full((HID, 2 * HID)), full((1, 2 * HID)),
            full((HID, 2 * HID)), full((1, 2 * HID)),
            full((HID, EW)), full((1, EW)),
        ],
        out_specs=[
            pl.BlockSpec((BLK, 3 * EMB), lambda i: (i, 0)),
            pl.BlockSpec((BLK, 1), lambda i: (i, 0)),
        ],
        out_shape=[
            jax.ShapeDtypeStruct((N, 3 * EMB), jnp.float32),
            jax.ShapeDtypeStruct((N, 1), jnp.float32),
        ],
    )(num, den3, hp1, hp2, bg, wzh1, bzh1, wzh2, bzh2, wef, bef)


def kernel(x, edge_index, h_prev1, h_prev2, W_gat, attn_l, attn_r, b_gat,
           Wz1, bz1, Wh1, bh1, Wz2, bz2, Wh2, bh2,
           We1, be1, We2, be2, We3, be3, Wf, bf):
    al = attn_l.reshape(1, HID)
    ar = attn_r.reshape(1, HID)
    h, a1k, a2k = _pre_call(x, W_gat, al, ar)

    num, den = _sc_edge(edge_index.reshape(2 * E), a1k.reshape(N),
                        a2k.reshape(N), h)

    wzh1 = jnp.concatenate([Wz1, Wh1], axis=1)
    bzh1 = jnp.concatenate([bz1, bh1]).reshape(1, 2 * HID)
    wzh2 = jnp.concatenate([Wz2, Wh2], axis=1)
    bzh2 = jnp.concatenate([bz2, bh2]).reshape(1, 2 * HID)
    wef = jnp.concatenate([We1, We2, We3, Wf], axis=1)
    bef = jnp.concatenate([be1, be2, be3, bf]).reshape(1, EW)
    cat, tp = _post_call(
        num, den.reshape(NC, NP, 1), h_prev1, h_prev2, b_gat.reshape(1, HID),
        wzh1, bzh1, wzh2, bzh2, wef, bef)
    return (cat.reshape(N, 3, EMB), tp.reshape(N))
